# Initial kernel scaffold; baseline (speedup 1.0000x reference)
import jax
import jax.numpy as jnp
from jax.experimental import pallas as pl


def kernel(query_states, W_qproj, Wq, Wk, Wv, Wo, W_out, ln_g, ln_b, memory_keys, memory_values):
    raise NotImplementedError("write your pallas kernel here")


# speedup vs baseline: 2.0283x; 2.0283x over previous
import functools
from typing import Any

import jax
import jax.numpy as jnp
import numpy as np
from jax.experimental import pallas as pl
from jax.experimental.pallas import tpu as pltpu

B, N, QD, MD, S, H, K, KW = 8, 256, 512, 512, 16384, 8, 32, 8
DH = MD // H
ALPHA, THRESH, UPD, DECAY, TEMP = 0.1, 1.0, 0.1, 0.001, 1.0
T = B * N


def _l2norm(x):
    return x / (jnp.linalg.norm(x, axis=-1, keepdims=True) + 1e-8)


def _sim_body(mq_ref, ck_ref, out_ref):
    out_ref[...] = jax.lax.dot_general(
        mq_ref[...], ck_ref[...], (((1,), (1,)), ((), ())),
        preferred_element_type=jnp.float32)


def _sim_matmul(mq, ck):
    TT, ST = 256, 2048
    return pl.pallas_call(
        _sim_body,
        grid=(T // TT, S // ST),
        in_specs=[
            pl.BlockSpec((TT, MD), lambda i, j: (i, 0)),
            pl.BlockSpec((ST, MD), lambda i, j: (j, 0)),
        ],
        out_specs=pl.BlockSpec((TT, ST), lambda i, j: (i, j)),
        out_shape=jax.ShapeDtypeStruct((T, S), jnp.float32),
    )(mq, ck)


def kernel(query_states, W_qproj, Wq, Wk, Wv, Wo, W_out, ln_g, ln_b,
           memory_keys, memory_values):
    qs = query_states.reshape(T, QD)
    mq = qs @ W_qproj
    ck = _l2norm(memory_keys)
    cv = _l2norm(memory_values)
    kW = ck @ Wk
    vW = cv @ Wv
    q = (mq @ Wq).reshape(T, H, DH)

    sim = _sim_matmul(mq, ck)
    _, top_idx = jax.lax.top_k(sim, K)

    khat = jnp.take(kW, top_idx, axis=0).reshape(T, K, H, DH)
    vhat = jnp.take(vW, top_idx, axis=0).reshape(T, K, H, DH)
    logits = jnp.einsum('thd,tkhd->thk', q, khat) / (np.sqrt(DH) * TEMP)
    attn = jax.nn.softmax(logits, axis=-1)
    ctx = jnp.einsum('thk,tkhd->thd', attn, vhat).reshape(T, MD)

    W2 = Wo @ W_out
    y = ctx @ W2
    mu = jnp.mean(y, axis=-1, keepdims=True)
    var = jnp.var(y, axis=-1, keepdims=True)
    out = ((y - mu) / jnp.sqrt(var + 1e-5) * ln_g + ln_b).reshape(B, N, QD)

    write_keys = mq.reshape(B, N, MD).mean(axis=1)
    wsim = write_keys @ memory_keys.T
    w_vals, w_idx = jax.lax.top_k(wsim, KW)
    comp = jax.nn.softmax(w_vals, axis=-1)
    w = comp * (0.5 * UPD)
    wf = w.reshape(B * KW)
    idxf = w_idx.reshape(B * KW)
    wkf = jnp.repeat(write_keys, KW, axis=0)

    onehot = (idxf[None, :] == jnp.arange(S)[:, None]).astype(jnp.float32)
    s_row = onehot @ wf
    a_row = onehot @ (wf[:, None] * wkf)
    upd_k = (memory_keys * (1.0 - s_row[:, None]) + a_row) * (1.0 - DECAY)
    upd_v = (memory_values * (1.0 - s_row[:, None]) + a_row) * (1.0 - DECAY)
    return (out, upd_k, upd_v)


# speedup vs baseline: 4.0895x; 2.0162x over previous
import functools

import jax
import jax.numpy as jnp
import numpy as np
from jax import lax
from jax.experimental import pallas as pl
from jax.experimental.pallas import tpu as pltpu
from jax.experimental.pallas import tpu_sc as plsc

B, N, QD, MD, S, H, K, KW = 8, 256, 512, 512, 16384, 8, 32, 8
DH = MD // H
ALPHA, THRESH, UPD, DECAY, TEMP = 0.1, 1.0, 0.1, 0.001, 1.0
T = B * N
TK = T * K
NEG = np.float32(-3.0e38)


def _mm_body(norm, a_ref, b_ref, o_ref):
    a = a_ref[...]
    if norm:
        a = a / (jnp.sqrt(jnp.sum(a * a, axis=1, keepdims=True)) + 1e-8)
    o_ref[...] = jnp.dot(a, b_ref[...], preferred_element_type=jnp.float32)


def _mm(a, b, tm, norm=False):
    m, kd = a.shape
    nd = b.shape[1]
    return pl.pallas_call(
        functools.partial(_mm_body, norm),
        grid=(m // tm,),
        in_specs=[
            pl.BlockSpec((tm, kd), lambda i: (i, 0)),
            pl.BlockSpec((kd, nd), lambda i: (0, 0)),
        ],
        out_specs=pl.BlockSpec((tm, nd), lambda i: (i, 0)),
        out_shape=jax.ShapeDtypeStruct((m, nd), jnp.float32),
    )(a, b)


def _sim_body(mq_ref, mk_ref, o_ref):
    ck = mk_ref[...]
    ck = ck / (jnp.sqrt(jnp.sum(ck * ck, axis=1, keepdims=True)) + 1e-8)
    o_ref[...] = lax.dot_general(
        mq_ref[...], ck, (((1,), (1,)), ((), ())),
        preferred_element_type=jnp.float32)


def _sim(mq, mk):
    TM, SN = 256, 2048
    return pl.pallas_call(
        _sim_body,
        grid=(S // SN, T // TM),
        in_specs=[
            pl.BlockSpec((TM, MD), lambda i, j: (j, 0)),
            pl.BlockSpec((SN, MD), lambda i, j: (i, 0)),
        ],
        out_specs=pl.BlockSpec((TM, SN), lambda i, j: (j, i)),
        out_shape=jax.ShapeDtypeStruct((T, S), jnp.float32),
    )(mq, mk)


def _topk_body(rows, nk, sim_ref, idx_ref):
    vals = sim_ref[...]
    iota = lax.broadcasted_iota(jnp.int32, (rows, S), 1)
    lane = lax.broadcasted_iota(jnp.int32, (rows, 128), 1)

    def step(k, carry):
        vals, acc = carry
        m = jnp.max(vals, axis=1, keepdims=True)
        cand = jnp.where(vals == m, iota, jnp.int32(S))
        ik = jnp.min(cand, axis=1, keepdims=True)
        vals = jnp.where(iota == ik, NEG, vals)
        acc = jnp.where(lane == k, ik, acc)
        return vals, acc

    _, acc = lax.fori_loop(
        0, nk, step, (vals, jnp.zeros((rows, 128), jnp.int32)))
    idx_ref[...] = acc


def _topk_idx(sim):
    TT = 64
    return pl.pallas_call(
        functools.partial(_topk_body, TT, K),
        grid=(T // TT,),
        in_specs=[pl.BlockSpec((TT, S), lambda i: (i, 0))],
        out_specs=pl.BlockSpec((TT, 128), lambda i: (i, 0)),
        out_shape=jax.ShapeDtypeStruct((T, 128), jnp.int32),
    )(sim)


def _gather_rows(kw, vw, idx):
    info = plsc.get_sparse_core_info()
    nc, ns = info.num_cores, info.num_subcores
    nw = nc * ns
    pw = TK // nw
    gc = 64

    mesh = plsc.VectorSubcoreMesh(
        core_axis_name="c", subcore_axis_name="s", num_cores=nc)

    @functools.partial(
        pl.kernel, mesh=mesh,
        out_type=(jax.ShapeDtypeStruct((TK, MD), jnp.float32),
                  jax.ShapeDtypeStruct((TK, MD), jnp.float32)),
        scratch_types=[
            pltpu.VMEM((gc,), jnp.int32),
            pltpu.VMEM((gc, MD), jnp.float32),
            pltpu.VMEM((gc, MD), jnp.float32),
            pltpu.SemaphoreType.DMA,
            pltpu.SemaphoreType.DMA,
        ],
    )
    def gk(kw_hbm, vw_hbm, idx_hbm, ok_hbm, ov_hbm, idx_v, rk_v, rv_v, sk, sv):
        wid = lax.axis_index("s") * nc + lax.axis_index("c")
        base = wid * pw

        def body(c, _):
            off = base + c * gc
            pltpu.sync_copy(idx_hbm.at[pl.ds(off, gc)], idx_v)
            a = pltpu.async_copy(kw_hbm.at[idx_v], rk_v, sk)
            b = pltpu.async_copy(vw_hbm.at[idx_v], rv_v, sv)
            a.wait()
            b.wait()
            pltpu.sync_copy(rk_v, ok_hbm.at[pl.ds(off, gc)])
            pltpu.sync_copy(rv_v, ov_hbm.at[pl.ds(off, gc)])
            return 0

        lax.fori_loop(0, pw // gc, body, 0)

    return gk(kw, vw, idx)


def _attn_body(q_ref, kh_ref, vh_ref, o_ref):
    scale = jnp.float32(1.0 / (np.sqrt(DH) * TEMP))
    for h in range(H):
        sl = slice(h * DH, (h + 1) * DH)
        qh = q_ref[:, sl]
        khh = kh_ref[:, :, sl]
        vhh = vh_ref[:, :, sl]
        logits = jnp.sum(qh[:, None, :] * khh, axis=2) * scale
        m = jnp.max(logits, axis=1, keepdims=True)
        e = jnp.exp(logits - m)
        a = e / jnp.sum(e, axis=1, keepdims=True)
        o_ref[:, sl] = jnp.sum(a[:, :, None] * vhh, axis=1)


def _attn(q, kh, vh):
    TT = 128
    return pl.pallas_call(
        _attn_body,
        grid=(T // TT,),
        in_specs=[
            pl.BlockSpec((TT, MD), lambda i: (i, 0)),
            pl.BlockSpec((TT, K, MD), lambda i: (i, 0, 0)),
            pl.BlockSpec((TT, K, MD), lambda i: (i, 0, 0)),
        ],
        out_specs=pl.BlockSpec((TT, MD), lambda i: (i, 0)),
        out_shape=jax.ShapeDtypeStruct((T, MD), jnp.float32),
    )(q, kh, vh)


def _out_body(a_ref, w_ref, g_ref, b_ref, o_ref):
    y = jnp.dot(a_ref[...], w_ref[...], preferred_element_type=jnp.float32)
    mu = jnp.mean(y, axis=1, keepdims=True)
    var = jnp.mean((y - mu) * (y - mu), axis=1, keepdims=True)
    o_ref[...] = (y - mu) / jnp.sqrt(var + 1e-5) * g_ref[...] + b_ref[...]


def _out_ln(ctx, w2, g, b):
    return pl.pallas_call(
        _out_body,
        grid=(1,),
        in_specs=[
            pl.BlockSpec((T, MD), lambda i: (0, 0)),
            pl.BlockSpec((MD, QD), lambda i: (0, 0)),
            pl.BlockSpec((1, QD), lambda i: (0, 0)),
            pl.BlockSpec((1, QD), lambda i: (0, 0)),
        ],
        out_specs=pl.BlockSpec((T, QD), lambda i: (0, 0)),
        out_shape=jax.ShapeDtypeStruct((T, QD), jnp.float32),
    )(ctx, w2, g.reshape(1, QD), b.reshape(1, QD))


def _wsim_body(mq_ref, mk_ref, ws_ref, wk_ref):
    wk = jnp.mean(mq_ref[...].reshape(B, N, MD), axis=1)
    wk_ref[...] = wk
    ws_ref[...] = lax.dot_general(
        wk, mk_ref[...], (((1,), (1,)), ((), ())),
        preferred_element_type=jnp.float32)


def _wsim(mq, mk):
    SN = 2048
    return pl.pallas_call(
        _wsim_body,
        grid=(S // SN,),
        in_specs=[
            pl.BlockSpec((T, MD), lambda i: (0, 0)),
            pl.BlockSpec((SN, MD), lambda i: (i, 0)),
        ],
        out_specs=[
            pl.BlockSpec((B, SN), lambda i: (0, i)),
            pl.BlockSpec((B, MD), lambda i: (0, 0)),
        ],
        out_shape=[jax.ShapeDtypeStruct((B, S), jnp.float32),
                   jax.ShapeDtypeStruct((B, MD), jnp.float32)],
    )(mq, mk)


def _wtopk_body(ws_ref, wi_ref, ww_ref):
    vals = ws_ref[...]
    iota = lax.broadcasted_iota(jnp.int32, (B, S), 1)
    lane = lax.broadcasted_iota(jnp.int32, (B, 128), 1)
    acc_i = jnp.zeros((B, 128), jnp.int32)
    acc_v = jnp.full((B, 128), NEG, jnp.float32)
    for k in range(KW):
        m = jnp.max(vals, axis=1, keepdims=True)
        cand = jnp.where(vals == m, iota, jnp.int32(S))
        ik = jnp.min(cand, axis=1, keepdims=True)
        vals = jnp.where(iota == ik, NEG, vals)
        acc_i = jnp.where(lane == k, ik, acc_i)
        acc_v = jnp.where(lane == k, m, acc_v)
    msk = lane < KW
    mx = jnp.max(acc_v, axis=1, keepdims=True)
    e = jnp.where(msk, jnp.exp(acc_v - mx), 0.0)
    w = e / jnp.sum(e, axis=1, keepdims=True) * jnp.float32(0.5 * UPD)
    ww_ref[...] = jnp.where(msk, w, 0.0)
    wi_ref[...] = acc_i


def _wtopk(ws):
    return pl.pallas_call(
        _wtopk_body,
        grid=(1,),
        in_specs=[pl.BlockSpec((B, S), lambda i: (0, 0))],
        out_specs=[pl.BlockSpec((B, 128), lambda i: (0, 0)),
                   pl.BlockSpec((B, 128), lambda i: (0, 0))],
        out_shape=[jax.ShapeDtypeStruct((B, 128), jnp.int32),
                   jax.ShapeDtypeStruct((B, 128), jnp.float32)],
    )(ws)


def _upd_body(mk_ref, mv_ref, wi_ref, ww_ref, wk_ref, ok_ref, ov_ref):
    sb = mk_ref.shape[0]
    i = pl.program_id(0)
    rows = i * sb + lax.broadcasted_iota(jnp.int32, (sb, B, KW), 0)
    idx = wi_ref[...][:, :KW]
    w = ww_ref[...][:, :KW]
    match = rows == idx[None, :, :]
    wb = jnp.sum(jnp.where(match, w[None, :, :], 0.0), axis=2)
    s_row = jnp.sum(wb, axis=1, keepdims=True)
    a_row = jnp.dot(wb, wk_ref[...], preferred_element_type=jnp.float32)
    dec = jnp.float32(1.0 - DECAY)
    ok_ref[...] = (mk_ref[...] * (1.0 - s_row) + a_row) * dec
    ov_ref[...] = (mv_ref[...] * (1.0 - s_row) + a_row) * dec


def _update(mk, mv, wi, ww, wk):
    SB = 2048
    return pl.pallas_call(
        _upd_body,
        grid=(S // SB,),
        in_specs=[
            pl.BlockSpec((SB, MD), lambda i: (i, 0)),
            pl.BlockSpec((SB, MD), lambda i: (i, 0)),
            pl.BlockSpec((B, 128), lambda i: (0, 0)),
            pl.BlockSpec((B, 128), lambda i: (0, 0)),
            pl.BlockSpec((B, MD), lambda i: (0, 0)),
        ],
        out_specs=[pl.BlockSpec((SB, MD), lambda i: (i, 0)),
                   pl.BlockSpec((SB, MD), lambda i: (i, 0))],
        out_shape=[jax.ShapeDtypeStruct((S, MD), jnp.float32),
                   jax.ShapeDtypeStruct((S, MD), jnp.float32)],
    )(mk, mv, wi, ww, wk)


def kernel(query_states, W_qproj, Wq, Wk, Wv, Wo, W_out, ln_g, ln_b,
           memory_keys, memory_values):
    qs = query_states.reshape(T, QD)
    mq = _mm(qs, W_qproj, 2048)
    q = _mm(mq, Wq, 2048)
    kw = _mm(memory_keys, Wk, 2048, norm=True)
    vw = _mm(memory_values, Wv, 2048, norm=True)
    w2 = _mm(Wo, W_out, 512)

    sim = _sim(mq, memory_keys)
    idx = _topk_idx(sim)[:, :K].reshape(TK)

    kh, vh = _gather_rows(kw, vw, idx)
    ctx = _attn(q, kh.reshape(T, K, MD), vh.reshape(T, K, MD))
    out = _out_ln(ctx, w2, ln_g, ln_b).reshape(B, N, QD)

    ws, wk = _wsim(mq, memory_keys)
    wi, ww = _wtopk(ws)
    upd_k, upd_v = _update(memory_keys, memory_values, wi, ww, wk)
    return (out, upd_k, upd_v)


# speedup vs baseline: 4.2463x; 1.0383x over previous
import functools

import jax
import jax.numpy as jnp
import numpy as np
from jax import lax
from jax.experimental import pallas as pl
from jax.experimental.pallas import tpu as pltpu
from jax.experimental.pallas import tpu_sc as plsc

B, N, QD, MD, S, H, K, KW = 8, 256, 512, 512, 16384, 8, 32, 8
DH = MD // H
ALPHA, THRESH, UPD, DECAY, TEMP = 0.1, 1.0, 0.1, 0.001, 1.0
T = B * N
TK = T * K
NEG = np.float32(-3.0e38)


def _mm_body(norm, a_ref, b_ref, o_ref):
    a = a_ref[...]
    if norm:
        a = a / (jnp.sqrt(jnp.sum(a * a, axis=1, keepdims=True)) + 1e-8)
    o_ref[...] = jnp.dot(a, b_ref[...], preferred_element_type=jnp.float32)


def _mm(a, b, tm, norm=False):
    m, kd = a.shape
    nd = b.shape[1]
    return pl.pallas_call(
        functools.partial(_mm_body, norm),
        grid=(m // tm,),
        in_specs=[
            pl.BlockSpec((tm, kd), lambda i: (i, 0)),
            pl.BlockSpec((kd, nd), lambda i: (0, 0)),
        ],
        out_specs=pl.BlockSpec((tm, nd), lambda i: (i, 0)),
        out_shape=jax.ShapeDtypeStruct((m, nd), jnp.float32),
    )(a, b)


def _sim_body(mq_ref, mk_ref, o_ref):
    ck = mk_ref[...]
    ck = ck / (jnp.sqrt(jnp.sum(ck * ck, axis=1, keepdims=True)) + 1e-8)
    o_ref[...] = lax.dot_general(
        mq_ref[...], ck, (((1,), (1,)), ((), ())),
        preferred_element_type=jnp.float32)


def _sim(mq, mk):
    SN = 2048
    return pl.pallas_call(
        _sim_body,
        grid=(S // SN,),
        in_specs=[
            pl.BlockSpec((T, MD), lambda i: (0, 0)),
            pl.BlockSpec((SN, MD), lambda i: (i, 0)),
        ],
        out_specs=pl.BlockSpec((T, SN), lambda i: (0, i)),
        out_shape=jax.ShapeDtypeStruct((T, S), jnp.float32),
    )(mq, mk)


def _topk_body(rows, nk, sim_ref, idx_ref):
    vals = sim_ref[...]
    iota = lax.broadcasted_iota(jnp.int32, (rows, S), 1)
    lane = lax.broadcasted_iota(jnp.int32, (rows, 128), 1)

    def step(k, carry):
        m_prev, ik_prev, acc = carry
        elig = (vals < m_prev) | ((vals == m_prev) & (iota > ik_prev))
        m = jnp.max(jnp.where(elig, vals, -jnp.inf), axis=1, keepdims=True)
        cand = jnp.where(elig & (vals == m), iota, jnp.int32(S))
        ik = jnp.min(cand, axis=1, keepdims=True)
        acc = jnp.where(lane == k, ik, acc)
        return m, ik, acc

    init = (jnp.full((rows, 1), jnp.inf, jnp.float32),
            jnp.full((rows, 1), -1, jnp.int32),
            jnp.zeros((rows, 128), jnp.int32))
    _, _, acc = lax.fori_loop(0, nk, step, init)
    idx_ref[...] = acc


def _topk_idx(sim):
    TT = 64
    return pl.pallas_call(
        functools.partial(_topk_body, TT, K),
        grid=(T // TT,),
        in_specs=[pl.BlockSpec((TT, S), lambda i: (i, 0))],
        out_specs=pl.BlockSpec((TT, 128), lambda i: (i, 0)),
        out_shape=jax.ShapeDtypeStruct((T, 128), jnp.int32),
    )(sim)


def _gather_rows(kw, vw, idx):
    info = plsc.get_sparse_core_info()
    nc, ns = info.num_cores, info.num_subcores
    nw = nc * ns
    pw = TK // nw
    gc = 64

    mesh = plsc.VectorSubcoreMesh(
        core_axis_name="c", subcore_axis_name="s", num_cores=nc)

    @functools.partial(
        pl.kernel, mesh=mesh,
        out_type=(jax.ShapeDtypeStruct((TK, MD), jnp.float32),
                  jax.ShapeDtypeStruct((TK, MD), jnp.float32)),
        scratch_types=[
            pltpu.VMEM((gc,), jnp.int32),
            pltpu.VMEM((gc, MD), jnp.float32),
            pltpu.VMEM((gc, MD), jnp.float32),
            pltpu.SemaphoreType.DMA,
            pltpu.SemaphoreType.DMA,
        ],
    )
    def gk(kw_hbm, vw_hbm, idx_hbm, ok_hbm, ov_hbm, idx_v, rk_v, rv_v, sk, sv):
        wid = lax.axis_index("s") * nc + lax.axis_index("c")
        base = wid * pw

        def body(c, _):
            off = base + c * gc
            pltpu.sync_copy(idx_hbm.at[pl.ds(off, gc)], idx_v)
            a = pltpu.async_copy(kw_hbm.at[idx_v], rk_v, sk)
            b = pltpu.async_copy(vw_hbm.at[idx_v], rv_v, sv)
            a.wait()
            b.wait()
            pltpu.sync_copy(rk_v, ok_hbm.at[pl.ds(off, gc)])
            pltpu.sync_copy(rv_v, ov_hbm.at[pl.ds(off, gc)])
            return 0

        lax.fori_loop(0, pw // gc, body, 0)

    return gk(kw, vw, idx)


def _attn_body(q_ref, kh_ref, vh_ref, o_ref):
    scale = jnp.float32(1.0 / (np.sqrt(DH) * TEMP))
    for h in range(H):
        sl = slice(h * DH, (h + 1) * DH)
        qh = q_ref[:, sl]
        khh = kh_ref[:, :, sl]
        vhh = vh_ref[:, :, sl]
        logits = jnp.sum(qh[:, None, :] * khh, axis=2) * scale
        m = jnp.max(logits, axis=1, keepdims=True)
        e = jnp.exp(logits - m)
        a = e / jnp.sum(e, axis=1, keepdims=True)
        o_ref[:, sl] = jnp.sum(a[:, :, None] * vhh, axis=1)


def _attn(q, kh, vh):
    TT = 128
    return pl.pallas_call(
        _attn_body,
        grid=(T // TT,),
        in_specs=[
            pl.BlockSpec((TT, MD), lambda i: (i, 0)),
            pl.BlockSpec((TT, K, MD), lambda i: (i, 0, 0)),
            pl.BlockSpec((TT, K, MD), lambda i: (i, 0, 0)),
        ],
        out_specs=pl.BlockSpec((TT, MD), lambda i: (i, 0)),
        out_shape=jax.ShapeDtypeStruct((T, MD), jnp.float32),
    )(q, kh, vh)


def _out_body(a_ref, w_ref, g_ref, b_ref, o_ref):
    y = jnp.dot(a_ref[...], w_ref[...], preferred_element_type=jnp.float32)
    mu = jnp.mean(y, axis=1, keepdims=True)
    var = jnp.mean((y - mu) * (y - mu), axis=1, keepdims=True)
    o_ref[...] = (y - mu) / jnp.sqrt(var + 1e-5) * g_ref[...] + b_ref[...]


def _out_ln(ctx, w2, g, b):
    return pl.pallas_call(
        _out_body,
        grid=(1,),
        in_specs=[
            pl.BlockSpec((T, MD), lambda i: (0, 0)),
            pl.BlockSpec((MD, QD), lambda i: (0, 0)),
            pl.BlockSpec((1, QD), lambda i: (0, 0)),
            pl.BlockSpec((1, QD), lambda i: (0, 0)),
        ],
        out_specs=pl.BlockSpec((T, QD), lambda i: (0, 0)),
        out_shape=jax.ShapeDtypeStruct((T, QD), jnp.float32),
    )(ctx, w2, g.reshape(1, QD), b.reshape(1, QD))


def _wsim_body(mq_ref, mk_ref, ws_ref, wk_ref):
    wk = jnp.mean(mq_ref[...].reshape(B, N, MD), axis=1)
    wk_ref[...] = wk
    ws_ref[...] = lax.dot_general(
        wk, mk_ref[...], (((1,), (1,)), ((), ())),
        preferred_element_type=jnp.float32)


def _wsim(mq, mk):
    SN = 2048
    return pl.pallas_call(
        _wsim_body,
        grid=(S // SN,),
        in_specs=[
            pl.BlockSpec((T, MD), lambda i: (0, 0)),
            pl.BlockSpec((SN, MD), lambda i: (i, 0)),
        ],
        out_specs=[
            pl.BlockSpec((B, SN), lambda i: (0, i)),
            pl.BlockSpec((B, MD), lambda i: (0, 0)),
        ],
        out_shape=[jax.ShapeDtypeStruct((B, S), jnp.float32),
                   jax.ShapeDtypeStruct((B, MD), jnp.float32)],
    )(mq, mk)


def _wtopk_body(ws_ref, wi_ref, ww_ref):
    vals = ws_ref[...]
    iota = lax.broadcasted_iota(jnp.int32, (B, S), 1)
    lane = lax.broadcasted_iota(jnp.int32, (B, 128), 1)
    acc_i = jnp.zeros((B, 128), jnp.int32)
    acc_v = jnp.full((B, 128), NEG, jnp.float32)
    m_prev = jnp.full((B, 1), jnp.inf, jnp.float32)
    ik_prev = jnp.full((B, 1), -1, jnp.int32)
    for k in range(KW):
        elig = (vals < m_prev) | ((vals == m_prev) & (iota > ik_prev))
        m = jnp.max(jnp.where(elig, vals, -jnp.inf), axis=1, keepdims=True)
        cand = jnp.where(elig & (vals == m), iota, jnp.int32(S))
        ik = jnp.min(cand, axis=1, keepdims=True)
        acc_i = jnp.where(lane == k, ik, acc_i)
        acc_v = jnp.where(lane == k, m, acc_v)
        m_prev, ik_prev = m, ik
    msk = lane < KW
    mx = jnp.max(acc_v, axis=1, keepdims=True)
    e = jnp.where(msk, jnp.exp(acc_v - mx), 0.0)
    w = e / jnp.sum(e, axis=1, keepdims=True) * jnp.float32(0.5 * UPD)
    ww_ref[...] = jnp.where(msk, w, 0.0)
    wi_ref[...] = acc_i


def _wtopk(ws):
    return pl.pallas_call(
        _wtopk_body,
        grid=(1,),
        in_specs=[pl.BlockSpec((B, S), lambda i: (0, 0))],
        out_specs=[pl.BlockSpec((B, 128), lambda i: (0, 0)),
                   pl.BlockSpec((B, 128), lambda i: (0, 0))],
        out_shape=[jax.ShapeDtypeStruct((B, 128), jnp.int32),
                   jax.ShapeDtypeStruct((B, 128), jnp.float32)],
    )(ws)


def _upd_body(mk_ref, mv_ref, wi_ref, ww_ref, wk_ref, ok_ref, ov_ref):
    sb = mk_ref.shape[0]
    i = pl.program_id(0)
    rows = i * sb + lax.broadcasted_iota(jnp.int32, (sb, B, KW), 0)
    idx = wi_ref[...][:, :KW]
    w = ww_ref[...][:, :KW]
    match = rows == idx[None, :, :]
    wb = jnp.sum(jnp.where(match, w[None, :, :], 0.0), axis=2)
    s_row = jnp.sum(wb, axis=1, keepdims=True)
    a_row = jnp.dot(wb, wk_ref[...], preferred_element_type=jnp.float32)
    dec = jnp.float32(1.0 - DECAY)
    ok_ref[...] = (mk_ref[...] * (1.0 - s_row) + a_row) * dec
    ov_ref[...] = (mv_ref[...] * (1.0 - s_row) + a_row) * dec


def _update(mk, mv, wi, ww, wk):
    SB = 2048
    return pl.pallas_call(
        _upd_body,
        grid=(S // SB,),
        in_specs=[
            pl.BlockSpec((SB, MD), lambda i: (i, 0)),
            pl.BlockSpec((SB, MD), lambda i: (i, 0)),
            pl.BlockSpec((B, 128), lambda i: (0, 0)),
            pl.BlockSpec((B, 128), lambda i: (0, 0)),
            pl.BlockSpec((B, MD), lambda i: (0, 0)),
        ],
        out_specs=[pl.BlockSpec((SB, MD), lambda i: (i, 0)),
                   pl.BlockSpec((SB, MD), lambda i: (i, 0))],
        out_shape=[jax.ShapeDtypeStruct((S, MD), jnp.float32),
                   jax.ShapeDtypeStruct((S, MD), jnp.float32)],
    )(mk, mv, wi, ww, wk)


def kernel(query_states, W_qproj, Wq, Wk, Wv, Wo, W_out, ln_g, ln_b,
           memory_keys, memory_values):
    qs = query_states.reshape(T, QD)
    mq = _mm(qs, W_qproj, 2048)
    q = _mm(mq, Wq, 2048)
    kw = _mm(memory_keys, Wk, 2048, norm=True)
    vw = _mm(memory_values, Wv, 2048, norm=True)
    w2 = _mm(Wo, W_out, 512)

    sim = _sim(mq, memory_keys)
    idx = _topk_idx(sim)[:, :K].reshape(TK)

    kh, vh = _gather_rows(kw, vw, idx)
    ctx = _attn(q, kh.reshape(T, K, MD), vh.reshape(T, K, MD))
    out = _out_ln(ctx, w2, ln_g, ln_b).reshape(B, N, QD)

    ws, wk = _wsim(mq, memory_keys)
    wi, ww = _wtopk(ws)
    upd_k, upd_v = _update(memory_keys, memory_values, wi, ww, wk)
    return (out, upd_k, upd_v)


# speedup vs baseline: 4.2666x; 1.0048x over previous
import functools

import jax
import jax.numpy as jnp
import numpy as np
from jax import lax
from jax.experimental import pallas as pl
from jax.experimental.pallas import tpu as pltpu
from jax.experimental.pallas import tpu_sc as plsc

B, N, QD, MD, S, H, K, KW = 8, 256, 512, 512, 16384, 8, 32, 8
DH = MD // H
ALPHA, THRESH, UPD, DECAY, TEMP = 0.1, 1.0, 0.1, 0.001, 1.0
T = B * N
TK = T * K
NEG = np.float32(-3.0e38)


def _mm_body(norm, a_ref, b_ref, o_ref):
    a = a_ref[...]
    if norm:
        a = a / (jnp.sqrt(jnp.sum(a * a, axis=1, keepdims=True)) + 1e-8)
    o_ref[...] = jnp.dot(a, b_ref[...], preferred_element_type=jnp.float32)


def _mm(a, b, tm, norm=False):
    m, kd = a.shape
    nd = b.shape[1]
    return pl.pallas_call(
        functools.partial(_mm_body, norm),
        grid=(m // tm,),
        in_specs=[
            pl.BlockSpec((tm, kd), lambda i: (i, 0)),
            pl.BlockSpec((kd, nd), lambda i: (0, 0)),
        ],
        out_specs=pl.BlockSpec((tm, nd), lambda i: (i, 0)),
        out_shape=jax.ShapeDtypeStruct((m, nd), jnp.float32),
    )(a, b)


def _sim_body(mq_ref, mk_ref, o_ref):
    ck = mk_ref[...]
    ck = ck / (jnp.sqrt(jnp.sum(ck * ck, axis=1, keepdims=True)) + 1e-8)
    o_ref[...] = lax.dot_general(
        mq_ref[...], ck, (((1,), (1,)), ((), ())),
        preferred_element_type=jnp.float32)


def _sim(mq, mk):
    SN = 2048
    return pl.pallas_call(
        _sim_body,
        grid=(S // SN,),
        in_specs=[
            pl.BlockSpec((T, MD), lambda i: (0, 0)),
            pl.BlockSpec((SN, MD), lambda i: (i, 0)),
        ],
        out_specs=pl.BlockSpec((T, SN), lambda i: (0, i)),
        out_shape=jax.ShapeDtypeStruct((T, S), jnp.float32),
    )(mq, mk)


def _topk_body(rows, nk, sim_ref, idx_ref):
    vals = sim_ref[...]
    iota = lax.broadcasted_iota(jnp.int32, (rows, S), 1)
    lane = lax.broadcasted_iota(jnp.int32, (rows, 128), 1)

    def step(k, carry):
        m_prev, ik_prev, acc = carry
        elig = (vals < m_prev) | ((vals == m_prev) & (iota > ik_prev))
        m = jnp.max(jnp.where(elig, vals, -jnp.inf), axis=1, keepdims=True)
        cand = jnp.where(elig & (vals == m), iota, jnp.int32(S))
        ik = jnp.min(cand, axis=1, keepdims=True)
        acc = jnp.where(lane == k, ik, acc)
        return m, ik, acc

    init = (jnp.full((rows, 1), jnp.inf, jnp.float32),
            jnp.full((rows, 1), -1, jnp.int32),
            jnp.zeros((rows, 128), jnp.int32))
    _, _, acc = lax.fori_loop(0, nk, step, init)
    idx_ref[...] = acc


def _topk_idx(sim):
    TT = 64
    return pl.pallas_call(
        functools.partial(_topk_body, TT, K),
        grid=(T // TT,),
        in_specs=[pl.BlockSpec((TT, S), lambda i: (i, 0))],
        out_specs=pl.BlockSpec((TT, 128), lambda i: (i, 0)),
        out_shape=jax.ShapeDtypeStruct((T, 128), jnp.int32),
    )(sim)


def _gather_rows(kw, vw, idx):
    info = plsc.get_sparse_core_info()
    nc, ns = info.num_cores, info.num_subcores
    nw = nc * ns
    pw = TK // nw
    gc = 32

    mesh = plsc.VectorSubcoreMesh(
        core_axis_name="c", subcore_axis_name="s", num_cores=nc)

    @functools.partial(
        pl.kernel, mesh=mesh,
        out_type=(jax.ShapeDtypeStruct((TK, MD), jnp.float32),
                  jax.ShapeDtypeStruct((TK, MD), jnp.float32)),
        scratch_types=[
            pltpu.VMEM((2, gc), jnp.int32),
            pltpu.VMEM((2, gc, MD), jnp.float32),
            pltpu.VMEM((2, gc, MD), jnp.float32),
            pltpu.SemaphoreType.DMA,
            pltpu.SemaphoreType.DMA,
            pltpu.SemaphoreType.DMA,
            pltpu.SemaphoreType.DMA,
        ],
    )
    def gk(kw_hbm, vw_hbm, idx_hbm, ok_hbm, ov_hbm, idx_v, rk_v, rv_v,
           sk0, sv0, sk1, sv1):
        wid = lax.axis_index("s") * nc + lax.axis_index("c")
        base = wid * pw
        nchunks = pw // gc
        sems = ((sk0, sv0), (sk1, sv1))

        def drain(b):
            pltpu.make_async_copy(
                kw_hbm.at[pl.ds(0, gc)], rk_v.at[b], sems[b][0]).wait()
            pltpu.make_async_copy(
                vw_hbm.at[pl.ds(0, gc)], rv_v.at[b], sems[b][1]).wait()

        def prefetch(off, b):
            pltpu.sync_copy(idx_hbm.at[pl.ds(off, gc)], idx_v.at[b])
            pltpu.async_copy(kw_hbm.at[idx_v.at[b]], rk_v.at[b], sems[b][0])
            pltpu.async_copy(vw_hbm.at[idx_v.at[b]], rv_v.at[b], sems[b][1])

        prefetch(base, 0)

        def body(c0, _):
            for b in range(2):
                c = c0 + b
                off = base + c * gc
                off_n = jnp.minimum(off + gc, TK - gc)
                prefetch(off_n, 1 - b)
                drain(b)
                pltpu.sync_copy(rk_v.at[b], ok_hbm.at[pl.ds(off, gc)])
                pltpu.sync_copy(rv_v.at[b], ov_hbm.at[pl.ds(off, gc)])
            return 0

        lax.fori_loop(0, nchunks // 2, lambda i, s: body(i * 2, s), 0)
        drain(nchunks % 2)

    return gk(kw, vw, idx)


def _attn_body(q_ref, kh_ref, vh_ref, o_ref):
    scale = jnp.float32(1.0 / (np.sqrt(DH) * TEMP))
    for h in range(H):
        sl = slice(h * DH, (h + 1) * DH)
        qh = q_ref[:, sl]
        khh = kh_ref[:, :, sl]
        vhh = vh_ref[:, :, sl]
        logits = jnp.sum(qh[:, None, :] * khh, axis=2) * scale
        m = jnp.max(logits, axis=1, keepdims=True)
        e = jnp.exp(logits - m)
        a = e / jnp.sum(e, axis=1, keepdims=True)
        o_ref[:, sl] = jnp.sum(a[:, :, None] * vhh, axis=1)


def _attn(q, kh, vh):
    TT = 128
    return pl.pallas_call(
        _attn_body,
        grid=(T // TT,),
        in_specs=[
            pl.BlockSpec((TT, MD), lambda i: (i, 0)),
            pl.BlockSpec((TT, K, MD), lambda i: (i, 0, 0)),
            pl.BlockSpec((TT, K, MD), lambda i: (i, 0, 0)),
        ],
        out_specs=pl.BlockSpec((TT, MD), lambda i: (i, 0)),
        out_shape=jax.ShapeDtypeStruct((T, MD), jnp.float32),
    )(q, kh, vh)


def _out_body(a_ref, w_ref, g_ref, b_ref, o_ref):
    y = jnp.dot(a_ref[...], w_ref[...], preferred_element_type=jnp.float32)
    mu = jnp.mean(y, axis=1, keepdims=True)
    var = jnp.mean((y - mu) * (y - mu), axis=1, keepdims=True)
    o_ref[...] = (y - mu) / jnp.sqrt(var + 1e-5) * g_ref[...] + b_ref[...]


def _out_ln(ctx, w2, g, b):
    return pl.pallas_call(
        _out_body,
        grid=(1,),
        in_specs=[
            pl.BlockSpec((T, MD), lambda i: (0, 0)),
            pl.BlockSpec((MD, QD), lambda i: (0, 0)),
            pl.BlockSpec((1, QD), lambda i: (0, 0)),
            pl.BlockSpec((1, QD), lambda i: (0, 0)),
        ],
        out_specs=pl.BlockSpec((T, QD), lambda i: (0, 0)),
        out_shape=jax.ShapeDtypeStruct((T, QD), jnp.float32),
    )(ctx, w2, g.reshape(1, QD), b.reshape(1, QD))


def _wsim_body(mq_ref, mk_ref, ws_ref, wk_ref):
    wk = jnp.mean(mq_ref[...].reshape(B, N, MD), axis=1)
    wk_ref[...] = wk
    ws_ref[...] = lax.dot_general(
        wk, mk_ref[...], (((1,), (1,)), ((), ())),
        preferred_element_type=jnp.float32)


def _wsim(mq, mk):
    SN = 2048
    return pl.pallas_call(
        _wsim_body,
        grid=(S // SN,),
        in_specs=[
            pl.BlockSpec((T, MD), lambda i: (0, 0)),
            pl.BlockSpec((SN, MD), lambda i: (i, 0)),
        ],
        out_specs=[
            pl.BlockSpec((B, SN), lambda i: (0, i)),
            pl.BlockSpec((B, MD), lambda i: (0, 0)),
        ],
        out_shape=[jax.ShapeDtypeStruct((B, S), jnp.float32),
                   jax.ShapeDtypeStruct((B, MD), jnp.float32)],
    )(mq, mk)


def _wtopk_body(ws_ref, wi_ref, ww_ref):
    vals = ws_ref[...]
    iota = lax.broadcasted_iota(jnp.int32, (B, S), 1)
    lane = lax.broadcasted_iota(jnp.int32, (B, 128), 1)
    acc_i = jnp.zeros((B, 128), jnp.int32)
    acc_v = jnp.full((B, 128), NEG, jnp.float32)
    m_prev = jnp.full((B, 1), jnp.inf, jnp.float32)
    ik_prev = jnp.full((B, 1), -1, jnp.int32)
    for k in range(KW):
        elig = (vals < m_prev) | ((vals == m_prev) & (iota > ik_prev))
        m = jnp.max(jnp.where(elig, vals, -jnp.inf), axis=1, keepdims=True)
        cand = jnp.where(elig & (vals == m), iota, jnp.int32(S))
        ik = jnp.min(cand, axis=1, keepdims=True)
        acc_i = jnp.where(lane == k, ik, acc_i)
        acc_v = jnp.where(lane == k, m, acc_v)
        m_prev, ik_prev = m, ik
    msk = lane < KW
    mx = jnp.max(acc_v, axis=1, keepdims=True)
    e = jnp.where(msk, jnp.exp(acc_v - mx), 0.0)
    w = e / jnp.sum(e, axis=1, keepdims=True) * jnp.float32(0.5 * UPD)
    ww_ref[...] = jnp.where(msk, w, 0.0)
    wi_ref[...] = acc_i


def _wtopk(ws):
    return pl.pallas_call(
        _wtopk_body,
        grid=(1,),
        in_specs=[pl.BlockSpec((B, S), lambda i: (0, 0))],
        out_specs=[pl.BlockSpec((B, 128), lambda i: (0, 0)),
                   pl.BlockSpec((B, 128), lambda i: (0, 0))],
        out_shape=[jax.ShapeDtypeStruct((B, 128), jnp.int32),
                   jax.ShapeDtypeStruct((B, 128), jnp.float32)],
    )(ws)


def _upd_body(mk_ref, mv_ref, wi_ref, ww_ref, wk_ref, ok_ref, ov_ref):
    sb = mk_ref.shape[0]
    i = pl.program_id(0)
    rows = i * sb + lax.broadcasted_iota(jnp.int32, (sb, B, KW), 0)
    idx = wi_ref[...][:, :KW]
    w = ww_ref[...][:, :KW]
    match = rows == idx[None, :, :]
    wb = jnp.sum(jnp.where(match, w[None, :, :], 0.0), axis=2)
    s_row = jnp.sum(wb, axis=1, keepdims=True)
    a_row = jnp.dot(wb, wk_ref[...], preferred_element_type=jnp.float32)
    dec = jnp.float32(1.0 - DECAY)
    ok_ref[...] = (mk_ref[...] * (1.0 - s_row) + a_row) * dec
    ov_ref[...] = (mv_ref[...] * (1.0 - s_row) + a_row) * dec


def _update(mk, mv, wi, ww, wk):
    SB = 2048
    return pl.pallas_call(
        _upd_body,
        grid=(S // SB,),
        in_specs=[
            pl.BlockSpec((SB, MD), lambda i: (i, 0)),
            pl.BlockSpec((SB, MD), lambda i: (i, 0)),
            pl.BlockSpec((B, 128), lambda i: (0, 0)),
            pl.BlockSpec((B, 128), lambda i: (0, 0)),
            pl.BlockSpec((B, MD), lambda i: (0, 0)),
        ],
        out_specs=[pl.BlockSpec((SB, MD), lambda i: (i, 0)),
                   pl.BlockSpec((SB, MD), lambda i: (i, 0))],
        out_shape=[jax.ShapeDtypeStruct((S, MD), jnp.float32),
                   jax.ShapeDtypeStruct((S, MD), jnp.float32)],
    )(mk, mv, wi, ww, wk)


def kernel(query_states, W_qproj, Wq, Wk, Wv, Wo, W_out, ln_g, ln_b,
           memory_keys, memory_values):
    qs = query_states.reshape(T, QD)
    mq = _mm(qs, W_qproj, 2048)
    q = _mm(mq, Wq, 2048)
    kw = _mm(memory_keys, Wk, 2048, norm=True)
    vw = _mm(memory_values, Wv, 2048, norm=True)
    w2 = _mm(Wo, W_out, 512)

    sim = _sim(mq, memory_keys)
    idx = _topk_idx(sim)[:, :K].reshape(TK)

    kh, vh = _gather_rows(kw, vw, idx)
    ctx = _attn(q, kh.reshape(T, K, MD), vh.reshape(T, K, MD))
    out = _out_ln(ctx, w2, ln_g, ln_b).reshape(B, N, QD)

    ws, wk = _wsim(mq, memory_keys)
    wi, ww = _wtopk(ws)
    upd_k, upd_v = _update(memory_keys, memory_values, wi, ww, wk)
    return (out, upd_k, upd_v)


# speedup vs baseline: 5.6390x; 1.3217x over previous
import functools

import jax
import jax.numpy as jnp
import numpy as np
from jax import lax
from jax.experimental import pallas as pl
from jax.experimental.pallas import tpu as pltpu
from jax.experimental.pallas import tpu_sc as plsc

B, N, QD, MD, S, H, K, KW = 8, 256, 512, 512, 16384, 8, 32, 8
DH = MD // H
ALPHA, THRESH, UPD, DECAY, TEMP = 0.1, 1.0, 0.1, 0.001, 1.0
T = B * N
TK = T * K
NEG = np.float32(-3.0e38)


def _mm_body(norm, a_ref, b_ref, o_ref):
    a = a_ref[...]
    if norm:
        a = a / (jnp.sqrt(jnp.sum(a * a, axis=1, keepdims=True)) + 1e-8)
    o_ref[...] = jnp.dot(a, b_ref[...], preferred_element_type=jnp.float32)


def _mm(a, b, tm, norm=False):
    m, kd = a.shape
    nd = b.shape[1]
    return pl.pallas_call(
        functools.partial(_mm_body, norm),
        grid=(m // tm,),
        in_specs=[
            pl.BlockSpec((tm, kd), lambda i: (i, 0)),
            pl.BlockSpec((kd, nd), lambda i: (0, 0)),
        ],
        out_specs=pl.BlockSpec((tm, nd), lambda i: (i, 0)),
        out_shape=jax.ShapeDtypeStruct((m, nd), jnp.float32),
    )(a, b)


def _sim_body(mq_ref, mk_ref, o_ref):
    ck = mk_ref[...]
    ck = ck / (jnp.sqrt(jnp.sum(ck * ck, axis=1, keepdims=True)) + 1e-8)
    o_ref[...] = lax.dot_general(
        mq_ref[...], ck, (((1,), (1,)), ((), ())),
        preferred_element_type=jnp.float32)


def _sim(mq, mk):
    SN = 2048
    return pl.pallas_call(
        _sim_body,
        grid=(S // SN,),
        in_specs=[
            pl.BlockSpec((T, MD), lambda i: (0, 0)),
            pl.BlockSpec((SN, MD), lambda i: (i, 0)),
        ],
        out_specs=pl.BlockSpec((T, SN), lambda i: (0, i)),
        out_shape=jax.ShapeDtypeStruct((T, S), jnp.float32),
    )(mq, mk)


def _topk_body(rows, nk, sim_ref, idx_ref):
    vals = sim_ref[...]
    msb = jnp.int32(-2147483648)
    bits = lax.bitcast_convert_type(vals, jnp.int32)
    ks = jnp.where(bits >= 0, bits, jnp.bitwise_xor(~bits, msb))
    iota = lax.broadcasted_iota(jnp.int32, (rows, S), 1)
    lane = lax.broadcasted_iota(jnp.int32, (rows, 128), 1)

    def rstep(i, p):
        cand = p | jnp.left_shift(jnp.int32(1), 31 - i)
        thr = jnp.bitwise_xor(cand, msb)
        cnt = jnp.sum((ks >= thr).astype(jnp.int32), axis=1, keepdims=True)
        return jnp.where(cnt >= nk, cand, p)

    p = lax.fori_loop(0, 32, rstep, jnp.zeros((rows, 1), jnp.int32))
    t = jnp.bitwise_xor(p, msb)

    cand0 = jnp.where(ks > t, iota,
                      jnp.where(ks == t, iota + S, jnp.int32(2 * S)))

    def estep(k, carry):
        m_prev, acc = carry
        m = jnp.min(jnp.where(cand0 > m_prev, cand0, jnp.int32(2 * S)),
                    axis=1, keepdims=True)
        iv = jnp.where(m >= S, m - S, m)
        acc = jnp.where(lane == k, iv, acc)
        return m, acc

    _, acc = lax.fori_loop(
        0, nk, estep,
        (jnp.full((rows, 1), -1, jnp.int32),
         jnp.zeros((rows, 128), jnp.int32)))
    idx_ref[...] = acc


def _topk_idx(sim):
    TT = 64
    return pl.pallas_call(
        functools.partial(_topk_body, TT, K),
        grid=(T // TT,),
        in_specs=[pl.BlockSpec((TT, S), lambda i: (i, 0))],
        out_specs=pl.BlockSpec((TT, 128), lambda i: (i, 0)),
        out_shape=jax.ShapeDtypeStruct((T, 128), jnp.int32),
    )(sim)


def _gather_rows(kw, vw, idx):
    info = plsc.get_sparse_core_info()
    nc, ns = info.num_cores, info.num_subcores
    nw = nc * ns
    pw = TK // nw
    gc = 32

    mesh = plsc.VectorSubcoreMesh(
        core_axis_name="c", subcore_axis_name="s", num_cores=nc)

    @functools.partial(
        pl.kernel, mesh=mesh,
        out_type=(jax.ShapeDtypeStruct((TK, MD), jnp.float32),
                  jax.ShapeDtypeStruct((TK, MD), jnp.float32)),
        scratch_types=[
            pltpu.VMEM((2, gc), jnp.int32),
            pltpu.VMEM((2, gc, MD), jnp.float32),
            pltpu.VMEM((2, gc, MD), jnp.float32),
            pltpu.SemaphoreType.DMA,
            pltpu.SemaphoreType.DMA,
            pltpu.SemaphoreType.DMA,
            pltpu.SemaphoreType.DMA,
        ],
    )
    def gk(kw_hbm, vw_hbm, idx_hbm, ok_hbm, ov_hbm, idx_v, rk_v, rv_v,
           sk0, sv0, sk1, sv1):
        wid = lax.axis_index("s") * nc + lax.axis_index("c")
        base = wid * pw
        nchunks = pw // gc
        sems = ((sk0, sv0), (sk1, sv1))

        def drain(b):
            pltpu.make_async_copy(
                kw_hbm.at[pl.ds(0, gc)], rk_v.at[b], sems[b][0]).wait()
            pltpu.make_async_copy(
                vw_hbm.at[pl.ds(0, gc)], rv_v.at[b], sems[b][1]).wait()

        def prefetch(off, b):
            pltpu.sync_copy(idx_hbm.at[pl.ds(off, gc)], idx_v.at[b])
            pltpu.async_copy(kw_hbm.at[idx_v.at[b]], rk_v.at[b], sems[b][0])
            pltpu.async_copy(vw_hbm.at[idx_v.at[b]], rv_v.at[b], sems[b][1])

        prefetch(base, 0)

        def body(c0, _):
            for b in range(2):
                c = c0 + b
                off = base + c * gc
                off_n = jnp.minimum(off + gc, TK - gc)
                prefetch(off_n, 1 - b)
                drain(b)
                pltpu.sync_copy(rk_v.at[b], ok_hbm.at[pl.ds(off, gc)])
                pltpu.sync_copy(rv_v.at[b], ov_hbm.at[pl.ds(off, gc)])
            return 0

        lax.fori_loop(0, nchunks // 2, lambda i, s: body(i * 2, s), 0)
        drain(nchunks % 2)

    return gk(kw, vw, idx)


def _attn_body(q_ref, kh_ref, vh_ref, o_ref):
    scale = jnp.float32(1.0 / (np.sqrt(DH) * TEMP))
    for h in range(H):
        sl = slice(h * DH, (h + 1) * DH)
        qh = q_ref[:, sl]
        khh = kh_ref[:, :, sl]
        vhh = vh_ref[:, :, sl]
        logits = jnp.sum(qh[:, None, :] * khh, axis=2) * scale
        m = jnp.max(logits, axis=1, keepdims=True)
        e = jnp.exp(logits - m)
        a = e / jnp.sum(e, axis=1, keepdims=True)
        o_ref[:, sl] = jnp.sum(a[:, :, None] * vhh, axis=1)


def _attn(q, kh, vh):
    TT = 128
    return pl.pallas_call(
        _attn_body,
        grid=(T // TT,),
        in_specs=[
            pl.BlockSpec((TT, MD), lambda i: (i, 0)),
            pl.BlockSpec((TT, K, MD), lambda i: (i, 0, 0)),
            pl.BlockSpec((TT, K, MD), lambda i: (i, 0, 0)),
        ],
        out_specs=pl.BlockSpec((TT, MD), lambda i: (i, 0)),
        out_shape=jax.ShapeDtypeStruct((T, MD), jnp.float32),
    )(q, kh, vh)


def _out_body(a_ref, w_ref, g_ref, b_ref, o_ref):
    y = jnp.dot(a_ref[...], w_ref[...], preferred_element_type=jnp.float32)
    mu = jnp.mean(y, axis=1, keepdims=True)
    var = jnp.mean((y - mu) * (y - mu), axis=1, keepdims=True)
    o_ref[...] = (y - mu) / jnp.sqrt(var + 1e-5) * g_ref[...] + b_ref[...]


def _out_ln(ctx, w2, g, b):
    return pl.pallas_call(
        _out_body,
        grid=(1,),
        in_specs=[
            pl.BlockSpec((T, MD), lambda i: (0, 0)),
            pl.BlockSpec((MD, QD), lambda i: (0, 0)),
            pl.BlockSpec((1, QD), lambda i: (0, 0)),
            pl.BlockSpec((1, QD), lambda i: (0, 0)),
        ],
        out_specs=pl.BlockSpec((T, QD), lambda i: (0, 0)),
        out_shape=jax.ShapeDtypeStruct((T, QD), jnp.float32),
    )(ctx, w2, g.reshape(1, QD), b.reshape(1, QD))


def _wsim_body(mq_ref, mk_ref, ws_ref, wk_ref):
    wk = jnp.mean(mq_ref[...].reshape(B, N, MD), axis=1)
    wk_ref[...] = wk
    ws_ref[...] = lax.dot_general(
        wk, mk_ref[...], (((1,), (1,)), ((), ())),
        preferred_element_type=jnp.float32)


def _wsim(mq, mk):
    SN = 2048
    return pl.pallas_call(
        _wsim_body,
        grid=(S // SN,),
        in_specs=[
            pl.BlockSpec((T, MD), lambda i: (0, 0)),
            pl.BlockSpec((SN, MD), lambda i: (i, 0)),
        ],
        out_specs=[
            pl.BlockSpec((B, SN), lambda i: (0, i)),
            pl.BlockSpec((B, MD), lambda i: (0, 0)),
        ],
        out_shape=[jax.ShapeDtypeStruct((B, S), jnp.float32),
                   jax.ShapeDtypeStruct((B, MD), jnp.float32)],
    )(mq, mk)


def _wtopk_body(ws_ref, wi_ref, ww_ref):
    vals = ws_ref[...]
    iota = lax.broadcasted_iota(jnp.int32, (B, S), 1)
    lane = lax.broadcasted_iota(jnp.int32, (B, 128), 1)
    acc_i = jnp.zeros((B, 128), jnp.int32)
    acc_v = jnp.full((B, 128), NEG, jnp.float32)
    m_prev = jnp.full((B, 1), jnp.inf, jnp.float32)
    ik_prev = jnp.full((B, 1), -1, jnp.int32)
    for k in range(KW):
        elig = (vals < m_prev) | ((vals == m_prev) & (iota > ik_prev))
        m = jnp.max(jnp.where(elig, vals, -jnp.inf), axis=1, keepdims=True)
        cand = jnp.where(elig & (vals == m), iota, jnp.int32(S))
        ik = jnp.min(cand, axis=1, keepdims=True)
        acc_i = jnp.where(lane == k, ik, acc_i)
        acc_v = jnp.where(lane == k, m, acc_v)
        m_prev, ik_prev = m, ik
    msk = lane < KW
    mx = jnp.max(acc_v, axis=1, keepdims=True)
    e = jnp.where(msk, jnp.exp(acc_v - mx), 0.0)
    w = e / jnp.sum(e, axis=1, keepdims=True) * jnp.float32(0.5 * UPD)
    ww_ref[...] = jnp.where(msk, w, 0.0)
    wi_ref[...] = acc_i


def _wtopk(ws):
    return pl.pallas_call(
        _wtopk_body,
        grid=(1,),
        in_specs=[pl.BlockSpec((B, S), lambda i: (0, 0))],
        out_specs=[pl.BlockSpec((B, 128), lambda i: (0, 0)),
                   pl.BlockSpec((B, 128), lambda i: (0, 0))],
        out_shape=[jax.ShapeDtypeStruct((B, 128), jnp.int32),
                   jax.ShapeDtypeStruct((B, 128), jnp.float32)],
    )(ws)


def _upd_body(mk_ref, mv_ref, wi_ref, ww_ref, wk_ref, ok_ref, ov_ref):
    sb = mk_ref.shape[0]
    i = pl.program_id(0)
    rows = i * sb + lax.broadcasted_iota(jnp.int32, (sb, B, KW), 0)
    idx = wi_ref[...][:, :KW]
    w = ww_ref[...][:, :KW]
    match = rows == idx[None, :, :]
    wb = jnp.sum(jnp.where(match, w[None, :, :], 0.0), axis=2)
    s_row = jnp.sum(wb, axis=1, keepdims=True)
    a_row = jnp.dot(wb, wk_ref[...], preferred_element_type=jnp.float32)
    dec = jnp.float32(1.0 - DECAY)
    ok_ref[...] = (mk_ref[...] * (1.0 - s_row) + a_row) * dec
    ov_ref[...] = (mv_ref[...] * (1.0 - s_row) + a_row) * dec


def _update(mk, mv, wi, ww, wk):
    SB = 2048
    return pl.pallas_call(
        _upd_body,
        grid=(S // SB,),
        in_specs=[
            pl.BlockSpec((SB, MD), lambda i: (i, 0)),
            pl.BlockSpec((SB, MD), lambda i: (i, 0)),
            pl.BlockSpec((B, 128), lambda i: (0, 0)),
            pl.BlockSpec((B, 128), lambda i: (0, 0)),
            pl.BlockSpec((B, MD), lambda i: (0, 0)),
        ],
        out_specs=[pl.BlockSpec((SB, MD), lambda i: (i, 0)),
                   pl.BlockSpec((SB, MD), lambda i: (i, 0))],
        out_shape=[jax.ShapeDtypeStruct((S, MD), jnp.float32),
                   jax.ShapeDtypeStruct((S, MD), jnp.float32)],
    )(mk, mv, wi, ww, wk)


def kernel(query_states, W_qproj, Wq, Wk, Wv, Wo, W_out, ln_g, ln_b,
           memory_keys, memory_values):
    qs = query_states.reshape(T, QD)
    mq = _mm(qs, W_qproj, 2048)
    q = _mm(mq, Wq, 2048)
    kw = _mm(memory_keys, Wk, 2048, norm=True)
    vw = _mm(memory_values, Wv, 2048, norm=True)
    w2 = _mm(Wo, W_out, 512)

    sim = _sim(mq, memory_keys)
    idx = _topk_idx(sim)[:, :K].reshape(TK)

    kh, vh = _gather_rows(kw, vw, idx)
    ctx = _attn(q, kh.reshape(T, K, MD), vh.reshape(T, K, MD))
    out = _out_ln(ctx, w2, ln_g, ln_b).reshape(B, N, QD)

    ws, wk = _wsim(mq, memory_keys)
    wi, ww = _wtopk(ws)
    upd_k, upd_v = _update(memory_keys, memory_values, wi, ww, wk)
    return (out, upd_k, upd_v)


# speedup vs baseline: 5.6414x; 1.0004x over previous
import functools

import jax
import jax.numpy as jnp
import numpy as np
from jax import lax
from jax.experimental import pallas as pl
from jax.experimental.pallas import tpu as pltpu
from jax.experimental.pallas import tpu_sc as plsc

B, N, QD, MD, S, H, K, KW = 8, 256, 512, 512, 16384, 8, 32, 8
DH = MD // H
ALPHA, THRESH, UPD, DECAY, TEMP = 0.1, 1.0, 0.1, 0.001, 1.0
T = B * N
TK = T * K
NEG = np.float32(-3.0e38)
_PAR = pltpu.CompilerParams(dimension_semantics=("parallel",))


def _mm_body(norm, a_ref, b_ref, o_ref):
    a = a_ref[...]
    if norm:
        a = a / (jnp.sqrt(jnp.sum(a * a, axis=1, keepdims=True)) + 1e-8)
    o_ref[...] = jnp.dot(a, b_ref[...], preferred_element_type=jnp.float32)


def _mm(a, b, tm, norm=False):
    m, kd = a.shape
    nd = b.shape[1]
    return pl.pallas_call(
        functools.partial(_mm_body, norm),
        grid=(m // tm,),
        in_specs=[
            pl.BlockSpec((tm, kd), lambda i: (i, 0)),
            pl.BlockSpec((kd, nd), lambda i: (0, 0)),
        ],
        out_specs=pl.BlockSpec((tm, nd), lambda i: (i, 0)),
        out_shape=jax.ShapeDtypeStruct((m, nd), jnp.float32),
        compiler_params=_PAR,
    )(a, b)


def _sim_body(mq_ref, mk_ref, o_ref):
    ck = mk_ref[...]
    ck = ck / (jnp.sqrt(jnp.sum(ck * ck, axis=1, keepdims=True)) + 1e-8)
    o_ref[...] = lax.dot_general(
        mq_ref[...], ck, (((1,), (1,)), ((), ())),
        preferred_element_type=jnp.float32)


def _sim(mq, mk):
    SN = 2048
    return pl.pallas_call(
        _sim_body,
        grid=(S // SN,),
        in_specs=[
            pl.BlockSpec((T, MD), lambda i: (0, 0)),
            pl.BlockSpec((SN, MD), lambda i: (i, 0)),
        ],
        out_specs=pl.BlockSpec((T, SN), lambda i: (0, i)),
        out_shape=jax.ShapeDtypeStruct((T, S), jnp.float32),
        compiler_params=_PAR,
    )(mq, mk)


def _topk_body(rows, nk, sim_ref, idx_ref):
    vals = sim_ref[...]
    msb = jnp.int32(-2147483648)
    bits = lax.bitcast_convert_type(vals, jnp.int32)
    ks = jnp.where(bits >= 0, bits, jnp.bitwise_xor(~bits, msb))
    iota = lax.broadcasted_iota(jnp.int32, (rows, S), 1)
    lane = lax.broadcasted_iota(jnp.int32, (rows, 128), 1)

    def rstep(i, p):
        cand = p | jnp.left_shift(jnp.int32(1), 31 - i)
        thr = jnp.bitwise_xor(cand, msb)
        cnt = jnp.sum((ks >= thr).astype(jnp.int32), axis=1, keepdims=True)
        return jnp.where(cnt >= nk, cand, p)

    p = lax.fori_loop(0, 32, rstep, jnp.zeros((rows, 1), jnp.int32))
    t = jnp.bitwise_xor(p, msb)

    cand0 = jnp.where(ks > t, iota,
                      jnp.where(ks == t, iota + S, jnp.int32(2 * S)))

    def estep(k, carry):
        m_prev, acc = carry
        m = jnp.min(jnp.where(cand0 > m_prev, cand0, jnp.int32(2 * S)),
                    axis=1, keepdims=True)
        iv = jnp.where(m >= S, m - S, m)
        acc = jnp.where(lane == k, iv, acc)
        return m, acc

    _, acc = lax.fori_loop(
        0, nk, estep,
        (jnp.full((rows, 1), -1, jnp.int32),
         jnp.zeros((rows, 128), jnp.int32)))
    idx_ref[...] = acc


def _topk_idx(sim):
    TT = 64
    return pl.pallas_call(
        functools.partial(_topk_body, TT, K),
        grid=(T // TT,),
        in_specs=[pl.BlockSpec((TT, S), lambda i: (i, 0))],
        out_specs=pl.BlockSpec((TT, 128), lambda i: (i, 0)),
        out_shape=jax.ShapeDtypeStruct((T, 128), jnp.int32),
        compiler_params=_PAR,
    )(sim)


def _gather_rows(kw, vw, idx):
    info = plsc.get_sparse_core_info()
    nc, ns = info.num_cores, info.num_subcores
    nw = nc * ns
    pw = TK // nw
    gc = 32

    mesh = plsc.VectorSubcoreMesh(
        core_axis_name="c", subcore_axis_name="s", num_cores=nc)

    @functools.partial(
        pl.kernel, mesh=mesh,
        out_type=(jax.ShapeDtypeStruct((TK, MD), jnp.float32),
                  jax.ShapeDtypeStruct((TK, MD), jnp.float32)),
        scratch_types=[
            pltpu.VMEM((2, gc), jnp.int32),
            pltpu.VMEM((2, gc, MD), jnp.float32),
            pltpu.VMEM((2, gc, MD), jnp.float32),
            pltpu.SemaphoreType.DMA,
            pltpu.SemaphoreType.DMA,
            pltpu.SemaphoreType.DMA,
            pltpu.SemaphoreType.DMA,
        ],
    )
    def gk(kw_hbm, vw_hbm, idx_hbm, ok_hbm, ov_hbm, idx_v, rk_v, rv_v,
           sk0, sv0, sk1, sv1):
        wid = lax.axis_index("s") * nc + lax.axis_index("c")
        base = wid * pw
        nchunks = pw // gc
        sems = ((sk0, sv0), (sk1, sv1))

        def drain(b):
            pltpu.make_async_copy(
                kw_hbm.at[pl.ds(0, gc)], rk_v.at[b], sems[b][0]).wait()
            pltpu.make_async_copy(
                vw_hbm.at[pl.ds(0, gc)], rv_v.at[b], sems[b][1]).wait()

        def prefetch(off, b):
            pltpu.sync_copy(idx_hbm.at[pl.ds(off, gc)], idx_v.at[b])
            pltpu.async_copy(kw_hbm.at[idx_v.at[b]], rk_v.at[b], sems[b][0])
            pltpu.async_copy(vw_hbm.at[idx_v.at[b]], rv_v.at[b], sems[b][1])

        prefetch(base, 0)

        def body(c0, _):
            for b in range(2):
                c = c0 + b
                off = base + c * gc
                off_n = jnp.minimum(off + gc, TK - gc)
                prefetch(off_n, 1 - b)
                drain(b)
                pltpu.sync_copy(rk_v.at[b], ok_hbm.at[pl.ds(off, gc)])
                pltpu.sync_copy(rv_v.at[b], ov_hbm.at[pl.ds(off, gc)])
            return 0

        lax.fori_loop(0, nchunks // 2, lambda i, s: body(i * 2, s), 0)
        drain(nchunks % 2)

    return gk(kw, vw, idx)


def _attn_body(q_ref, kh_ref, vh_ref, o_ref):
    scale = jnp.float32(1.0 / (np.sqrt(DH) * TEMP))
    for h in range(H):
        sl = slice(h * DH, (h + 1) * DH)
        qh = q_ref[:, sl]
        khh = kh_ref[:, :, sl]
        vhh = vh_ref[:, :, sl]
        logits = jnp.sum(qh[:, None, :] * khh, axis=2) * scale
        m = jnp.max(logits, axis=1, keepdims=True)
        e = jnp.exp(logits - m)
        a = e / jnp.sum(e, axis=1, keepdims=True)
        o_ref[:, sl] = jnp.sum(a[:, :, None] * vhh, axis=1)


def _attn(q, kh, vh):
    TT = 128
    return pl.pallas_call(
        _attn_body,
        grid=(T // TT,),
        in_specs=[
            pl.BlockSpec((TT, MD), lambda i: (i, 0)),
            pl.BlockSpec((TT, K, MD), lambda i: (i, 0, 0)),
            pl.BlockSpec((TT, K, MD), lambda i: (i, 0, 0)),
        ],
        out_specs=pl.BlockSpec((TT, MD), lambda i: (i, 0)),
        out_shape=jax.ShapeDtypeStruct((T, MD), jnp.float32),
        compiler_params=_PAR,
    )(q, kh, vh)


def _out_body(a_ref, w_ref, g_ref, b_ref, o_ref):
    y = jnp.dot(a_ref[...], w_ref[...], preferred_element_type=jnp.float32)
    mu = jnp.mean(y, axis=1, keepdims=True)
    var = jnp.mean((y - mu) * (y - mu), axis=1, keepdims=True)
    o_ref[...] = (y - mu) / jnp.sqrt(var + 1e-5) * g_ref[...] + b_ref[...]


def _out_ln(ctx, w2, g, b):
    return pl.pallas_call(
        _out_body,
        grid=(1,),
        in_specs=[
            pl.BlockSpec((T, MD), lambda i: (0, 0)),
            pl.BlockSpec((MD, QD), lambda i: (0, 0)),
            pl.BlockSpec((1, QD), lambda i: (0, 0)),
            pl.BlockSpec((1, QD), lambda i: (0, 0)),
        ],
        out_specs=pl.BlockSpec((T, QD), lambda i: (0, 0)),
        out_shape=jax.ShapeDtypeStruct((T, QD), jnp.float32),
    )(ctx, w2, g.reshape(1, QD), b.reshape(1, QD))


def _wsim_body(mq_ref, mk_ref, ws_ref, wk_ref):
    wk = jnp.mean(mq_ref[...].reshape(B, N, MD), axis=1)
    wk_ref[...] = wk
    ws_ref[...] = lax.dot_general(
        wk, mk_ref[...], (((1,), (1,)), ((), ())),
        preferred_element_type=jnp.float32)


def _wsim(mq, mk):
    SN = 2048
    return pl.pallas_call(
        _wsim_body,
        grid=(S // SN,),
        in_specs=[
            pl.BlockSpec((T, MD), lambda i: (0, 0)),
            pl.BlockSpec((SN, MD), lambda i: (i, 0)),
        ],
        out_specs=[
            pl.BlockSpec((B, SN), lambda i: (0, i)),
            pl.BlockSpec((B, MD), lambda i: (0, 0)),
        ],
        out_shape=[jax.ShapeDtypeStruct((B, S), jnp.float32),
                   jax.ShapeDtypeStruct((B, MD), jnp.float32)],
    )(mq, mk)


def _wtopk_body(ws_ref, wi_ref, ww_ref):
    vals = ws_ref[...]
    iota = lax.broadcasted_iota(jnp.int32, (B, S), 1)
    lane = lax.broadcasted_iota(jnp.int32, (B, 128), 1)
    acc_i = jnp.zeros((B, 128), jnp.int32)
    acc_v = jnp.full((B, 128), NEG, jnp.float32)
    m_prev = jnp.full((B, 1), jnp.inf, jnp.float32)
    ik_prev = jnp.full((B, 1), -1, jnp.int32)
    for k in range(KW):
        elig = (vals < m_prev) | ((vals == m_prev) & (iota > ik_prev))
        m = jnp.max(jnp.where(elig, vals, -jnp.inf), axis=1, keepdims=True)
        cand = jnp.where(elig & (vals == m), iota, jnp.int32(S))
        ik = jnp.min(cand, axis=1, keepdims=True)
        acc_i = jnp.where(lane == k, ik, acc_i)
        acc_v = jnp.where(lane == k, m, acc_v)
        m_prev, ik_prev = m, ik
    msk = lane < KW
    mx = jnp.max(acc_v, axis=1, keepdims=True)
    e = jnp.where(msk, jnp.exp(acc_v - mx), 0.0)
    w = e / jnp.sum(e, axis=1, keepdims=True) * jnp.float32(0.5 * UPD)
    ww_ref[...] = jnp.where(msk, w, 0.0)
    wi_ref[...] = acc_i


def _wtopk(ws):
    return pl.pallas_call(
        _wtopk_body,
        grid=(1,),
        in_specs=[pl.BlockSpec((B, S), lambda i: (0, 0))],
        out_specs=[pl.BlockSpec((B, 128), lambda i: (0, 0)),
                   pl.BlockSpec((B, 128), lambda i: (0, 0))],
        out_shape=[jax.ShapeDtypeStruct((B, 128), jnp.int32),
                   jax.ShapeDtypeStruct((B, 128), jnp.float32)],
    )(ws)


def _upd_body(mk_ref, mv_ref, wi_ref, ww_ref, wk_ref, ok_ref, ov_ref):
    sb = mk_ref.shape[0]
    i = pl.program_id(0)
    rows = i * sb + lax.broadcasted_iota(jnp.int32, (sb, B, KW), 0)
    idx = wi_ref[...][:, :KW]
    w = ww_ref[...][:, :KW]
    match = rows == idx[None, :, :]
    wb = jnp.sum(jnp.where(match, w[None, :, :], 0.0), axis=2)
    s_row = jnp.sum(wb, axis=1, keepdims=True)
    a_row = jnp.dot(wb, wk_ref[...], preferred_element_type=jnp.float32)
    dec = jnp.float32(1.0 - DECAY)
    ok_ref[...] = (mk_ref[...] * (1.0 - s_row) + a_row) * dec
    ov_ref[...] = (mv_ref[...] * (1.0 - s_row) + a_row) * dec


def _update(mk, mv, wi, ww, wk):
    SB = 2048
    return pl.pallas_call(
        _upd_body,
        grid=(S // SB,),
        in_specs=[
            pl.BlockSpec((SB, MD), lambda i: (i, 0)),
            pl.BlockSpec((SB, MD), lambda i: (i, 0)),
            pl.BlockSpec((B, 128), lambda i: (0, 0)),
            pl.BlockSpec((B, 128), lambda i: (0, 0)),
            pl.BlockSpec((B, MD), lambda i: (0, 0)),
        ],
        out_specs=[pl.BlockSpec((SB, MD), lambda i: (i, 0)),
                   pl.BlockSpec((SB, MD), lambda i: (i, 0))],
        out_shape=[jax.ShapeDtypeStruct((S, MD), jnp.float32),
                   jax.ShapeDtypeStruct((S, MD), jnp.float32)],
    )(mk, mv, wi, ww, wk)


def kernel(query_states, W_qproj, Wq, Wk, Wv, Wo, W_out, ln_g, ln_b,
           memory_keys, memory_values):
    qs = query_states.reshape(T, QD)
    mq = _mm(qs, W_qproj, 2048)
    q = _mm(mq, Wq, 2048)
    kw = _mm(memory_keys, Wk, 2048, norm=True)
    vw = _mm(memory_values, Wv, 2048, norm=True)
    w2 = _mm(Wo, W_out, 512)

    sim = _sim(mq, memory_keys)
    idx = _topk_idx(sim)[:, :K].reshape(TK)

    kh, vh = _gather_rows(kw, vw, idx)
    ctx = _attn(q, kh.reshape(T, K, MD), vh.reshape(T, K, MD))
    out = _out_ln(ctx, w2, ln_g, ln_b).reshape(B, N, QD)

    ws, wk = _wsim(mq, memory_keys)
    wi, ww = _wtopk(ws)
    upd_k, upd_v = _update(memory_keys, memory_values, wi, ww, wk)
    return (out, upd_k, upd_v)
